# Initial kernel scaffold; baseline (speedup 1.0000x reference)
import jax
import jax.numpy as jnp
from jax.experimental import pallas as pl


def kernel(x, emb, W1, b1, W2, b2):
    raise NotImplementedError("write your pallas kernel here")


# speedup vs baseline: 10.2905x; 10.2905x over previous
import functools

import jax
import jax.numpy as jnp
from jax import lax
from jax.experimental import pallas as pl
from jax.experimental.pallas import tpu as pltpu
from jax.experimental.pallas import tpu_sc as plsc

_B = 16384
_L = 50
_D_E = 32
_D_H = 64
_D_OUT = 32
_N = _B * _L

_NC = 2
_NS = 16
_NW = _NC * _NS
_PER_W = _N // _NW
_CHUNK = 128
_NCHUNK = _PER_W // _CHUNK
_NBUF = 4
_NGROUP = _NCHUNK // _NBUF


def _sc_gather_body(emb_hbm, idx_hbm, out_hbm, idx_v, rows_v, *sems):
    gs = sems[:_NBUF]
    ws = sems[_NBUF:]
    wid = lax.axis_index("s") * _NC + lax.axis_index("c")
    base = wid * _PER_W

    pltpu.sync_copy(idx_hbm.at[wid], idx_v)

    def g_copy(c, b):
        return pltpu.make_async_copy(
            emb_hbm.at[idx_v.at[c]], rows_v.at[b], gs[b])

    def w_copy(c, b):
        return pltpu.make_async_copy(
            rows_v.at[b], out_hbm.at[pl.ds(base + c * _CHUNK, _CHUNK)], ws[b])

    for b in range(_NBUF):
        g_copy(b, b).start()

    def body(g, _):
        for b in range(_NBUF):
            c = g * _NBUF + b
            g_copy(c, b).wait()
            w_copy(c, b).start()

        @pl.when(g < _NGROUP - 1)
        def _prefetch():
            for b in range(_NBUF):
                c = (g + 1) * _NBUF + b
                w_copy(c, b).wait()
                g_copy(c, b).start()

        return _

    lax.fori_loop(0, _NGROUP, body, 0)

    last = (_NGROUP - 1) * _NBUF
    for b in range(_NBUF):
        w_copy(last + b, b).wait()


@functools.partial(jax.jit, static_argnums=())
def _sc_gather(emb, idx3):
    mesh = plsc.VectorSubcoreMesh(
        core_axis_name="c", subcore_axis_name="s",
        num_cores=_NC, num_subcores=_NS)
    return pl.kernel(
        _sc_gather_body,
        out_type=jax.ShapeDtypeStruct((_N, _D_E), jnp.float32),
        mesh=mesh,
        scratch_types=[
            pltpu.VMEM((_NCHUNK, _CHUNK), jnp.int32),
            pltpu.VMEM((_NBUF, _CHUNK, _D_E), jnp.float32),
        ] + [pltpu.SemaphoreType.DMA] * (2 * _NBUF),
        compiler_params=pltpu.CompilerParams(use_tc_tiling_on_sc=False),
    )(emb, idx3)


def _mlp_body(g_ref, w1_ref, b1_ref, w2_ref, b2_ref, o_ref):
    e = g_ref[...]
    h = jnp.dot(e, w1_ref[...], preferred_element_type=jnp.float32)
    h = jnp.maximum(h + b1_ref[...], 0.0)
    o = jnp.dot(h, w2_ref[...], preferred_element_type=jnp.float32)
    o_ref[...] = o + b2_ref[...]


_RB = 8192


def _mlp(g, W1, b1, W2, b2):
    grid = (_N // _RB,)
    return pl.pallas_call(
        _mlp_body,
        grid=grid,
        in_specs=[
            pl.BlockSpec((_RB, _D_E), lambda i: (i, 0)),
            pl.BlockSpec((_D_E, _D_H), lambda i: (0, 0)),
            pl.BlockSpec((1, _D_H), lambda i: (0, 0)),
            pl.BlockSpec((_D_H, _D_OUT), lambda i: (0, 0)),
            pl.BlockSpec((1, _D_OUT), lambda i: (0, 0)),
        ],
        out_specs=pl.BlockSpec((_RB, _D_OUT), lambda i: (i, 0)),
        out_shape=jax.ShapeDtypeStruct((_N, _D_OUT), jnp.float32),
    )(g, W1, b1.reshape(1, _D_H), W2, b2.reshape(1, _D_OUT))


def kernel(x, emb, W1, b1, W2, b2):
    idx3 = x.astype(jnp.int32).reshape(_NW, _NCHUNK, _CHUNK)
    g = _sc_gather(emb, idx3)
    out = _mlp(g, W1, b1, W2, b2)
    return out.reshape(_B, _L, _D_OUT)


# speedup vs baseline: 19.5413x; 1.8990x over previous
import functools

import jax
import jax.numpy as jnp
from jax import lax
from jax.experimental import pallas as pl
from jax.experimental.pallas import tpu as pltpu
from jax.experimental.pallas import tpu_sc as plsc

_B = 16384
_L = 50
_D_E = 32
_D_H = 64
_D_OUT = 32
_N = _B * _L

_NC = 2
_NS = 16
_NW = _NC * _NS
_PER_W = _N // _NW
_CHUNK = 128
_NCHUNK = _PER_W // _CHUNK
_NBUF = 4
_NGROUP = _NCHUNK // _NBUF
_IROW_W = _PER_W // _CHUNK


def _sc_gather_body(emb_hbm, idx_hbm, out_hbm, idx_v, rows_v, *sems):
    gs = sems[:_NBUF]
    ws = sems[_NBUF:]
    wid = lax.axis_index("s") * _NC + lax.axis_index("c")
    base = wid * _PER_W

    pltpu.sync_copy(idx_hbm.at[pl.ds(wid * _IROW_W, _IROW_W)], idx_v)

    def g_copy(c, b):
        return pltpu.make_async_copy(
            emb_hbm.at[idx_v.at[c]], rows_v.at[b], gs[b])

    def w_copy(c, b):
        return pltpu.make_async_copy(
            rows_v.at[b], out_hbm.at[pl.ds(base + c * _CHUNK, _CHUNK)], ws[b])

    for b in range(_NBUF):
        g_copy(b, b).start()

    def body(g, carry):
        for b in range(_NBUF):
            c = g * _NBUF + b
            g_copy(c, b).wait()
            w_copy(c, b).start()

        @pl.when(g < _NGROUP - 1)
        def _prefetch():
            for b in range(_NBUF):
                c = (g + 1) * _NBUF + b
                w_copy(c, b).wait()
                g_copy(c, b).start()

        return carry

    lax.fori_loop(0, _NGROUP, body, 0)

    last = (_NGROUP - 1) * _NBUF
    for b in range(_NBUF):
        w_copy(last + b, b).wait()


def _sc_gather(emb, idx2):
    mesh = plsc.VectorSubcoreMesh(
        core_axis_name="c", subcore_axis_name="s",
        num_cores=_NC, num_subcores=_NS)
    return pl.kernel(
        _sc_gather_body,
        out_type=jax.ShapeDtypeStruct((_N, _D_E), jnp.float32),
        mesh=mesh,
        scratch_types=[
            pltpu.VMEM((_NCHUNK, _CHUNK), jnp.int32),
            pltpu.VMEM((_NBUF, _CHUNK, _D_E), jnp.float32),
        ] + [pltpu.SemaphoreType.DMA] * (2 * _NBUF),
        compiler_params=pltpu.CompilerParams(use_tc_tiling_on_sc=False),
    )(emb, idx2)


def _mlp_body(g_ref, w1_ref, b1_ref, w2_ref, b2_ref, o_ref):
    e4 = g_ref[...]
    h4 = jnp.dot(e4, w1_ref[...], preferred_element_type=jnp.float32)
    h4 = jnp.maximum(h4 + b1_ref[...], 0.0)
    o4 = jnp.dot(h4, w2_ref[...], preferred_element_type=jnp.float32)
    o_ref[...] = o4 + b2_ref[...]


_GB = 3200


def _mlp(g4, W14, b14, W24, b24):
    return pl.pallas_call(
        _mlp_body,
        grid=(_N // 4 // _GB,),
        in_specs=[
            pl.BlockSpec((_GB, 128), lambda i: (i, 0)),
            pl.BlockSpec((128, 4 * _D_H), lambda i: (0, 0)),
            pl.BlockSpec((1, 4 * _D_H), lambda i: (0, 0)),
            pl.BlockSpec((4 * _D_H, 128), lambda i: (0, 0)),
            pl.BlockSpec((1, 128), lambda i: (0, 0)),
        ],
        out_specs=pl.BlockSpec((_GB, 128), lambda i: (i, 0)),
        out_shape=jax.ShapeDtypeStruct((_N // 4, 128), jnp.float32),
    )(g4, W14, b14, W24, b24)


def kernel(x, emb, W1, b1, W2, b2):
    idx2 = x.T.astype(jnp.int32).reshape(_N // _CHUNK, _CHUNK)
    g = _sc_gather(emb, idx2)
    g4 = g.reshape(_N // 4, 128)
    eye4 = jnp.eye(4, dtype=jnp.float32)
    W14 = jnp.kron(eye4, W1)
    W24 = jnp.kron(eye4, W2)
    b14 = jnp.tile(b1, 4).reshape(1, 4 * _D_H)
    b24 = jnp.tile(b2, 4).reshape(1, 128)
    o4 = _mlp(g4, W14, b14, W24, b24)
    return o4.reshape(_L, _B, _D_OUT).transpose(1, 0, 2)


# speedup vs baseline: 44.4554x; 2.2749x over previous
import functools

import jax
import jax.numpy as jnp
from jax import lax
from jax.experimental import pallas as pl
from jax.experimental.pallas import tpu as pltpu
from jax.experimental.pallas import tpu_sc as plsc

_B = 16384
_L = 50
_D_E = 32
_D_H = 64
_D_OUT = 32
_N = _B * _L
_V = 1000000

_NC = 2
_NS = 16
_NW = _NC * _NS
_PER_W = _N // _NW
_CHUNK = 128
_NCHUNK = _PER_W // _CHUNK
_NBUF = 4
_NGROUP = _NCHUNK // _NBUF
_IROW_W = _PER_W // _CHUNK

_PB = 8192
_PQ = _PB // 4
_PG = (_V + _PB - 1) // _PB
_VP = _PG * _PB


def _pack_body(e_ref, o_ref):
    et = e_ref[...].T
    o_ref[...] = jnp.concatenate(
        [et[0:_PQ], et[_PQ:2 * _PQ], et[2 * _PQ:3 * _PQ], et[3 * _PQ:4 * _PQ]],
        axis=1)


def _pack_table(emb_t):
    return pl.pallas_call(
        _pack_body,
        grid=(_PG,),
        in_specs=[pl.BlockSpec((_D_E, _PB), lambda c: (0, c))],
        out_specs=pl.BlockSpec((_PQ, 128), lambda c: (c, 0)),
        out_shape=jax.ShapeDtypeStruct((_VP // 4, 128), jnp.float32),
    )(emb_t)


def _sc_gather_body(tab_hbm, idx_hbm, out_hbm, idx_v, rows_v, dsti, *sems):
    gs = sems[:_NBUF]
    ws = sems[_NBUF:]
    wid = lax.axis_index("s") * _NC + lax.axis_index("c")
    base = wid * _PER_W
    iv16 = lax.iota(jnp.int32, 16)

    pltpu.sync_copy(idx_hbm.at[pl.ds(wid * _IROW_W, _IROW_W)], idx_v)

    def g_copy(c, b):
        return pltpu.make_async_copy(
            tab_hbm.at[idx_v.at[c]], rows_v.at[b], gs[b])

    def w_copy(b):
        return pltpu.make_async_copy(
            rows_v.at[b], out_hbm.at[dsti.at[b]], ws[b])

    def fill_dst(c, b):
        t0 = base + c * _CHUNK
        d0 = ((t0 & ~16383) + ((t0 & 4095) << 2) + ((t0 >> 12) & 3))
        for v in range(8):
            dsti[b, pl.ds(v * 16, 16)] = d0 + 4 * (v * 16 + iv16)

    for b in range(_NBUF):
        g_copy(b, b).start()

    def body(g, carry):
        for b in range(_NBUF):
            c = g * _NBUF + b
            g_copy(c, b).wait()
            fill_dst(c, b)
            w_copy(b).start()

        @pl.when(g < _NGROUP - 1)
        def _prefetch():
            for b in range(_NBUF):
                c = (g + 1) * _NBUF + b
                w_copy(b).wait()
                g_copy(c, b).start()

        return carry

    lax.fori_loop(0, _NGROUP, body, 0)

    for b in range(_NBUF):
        w_copy(b).wait()


def _sc_gather(tab, idx2):
    mesh = plsc.VectorSubcoreMesh(
        core_axis_name="c", subcore_axis_name="s",
        num_cores=_NC, num_subcores=_NS)
    return pl.kernel(
        _sc_gather_body,
        out_type=jax.ShapeDtypeStruct((_N, _D_E), jnp.float32),
        mesh=mesh,
        scratch_types=[
            pltpu.VMEM((_NCHUNK, _CHUNK), jnp.int32),
            pltpu.VMEM((_NBUF, _CHUNK, _D_E), jnp.float32),
            pltpu.VMEM((_NBUF, _CHUNK), jnp.int32),
        ] + [pltpu.SemaphoreType.DMA] * (2 * _NBUF),
        compiler_params=pltpu.CompilerParams(use_tc_tiling_on_sc=False),
    )(tab, idx2)


def _mlp_body(g_ref, w1t_ref, b1_ref, w2t_ref, b2_ref, o_ref):
    e4t = g_ref[...].T
    for q in range(4):
        e = e4t[32 * q:32 * (q + 1), :]
        h = jnp.dot(w1t_ref[...], e, preferred_element_type=jnp.float32)
        h = jnp.maximum(h + b1_ref[...], 0.0)
        o = jnp.dot(w2t_ref[...], h, preferred_element_type=jnp.float32)
        o_ref[0, :, 4096 * q:4096 * (q + 1)] = o + b2_ref[...]


def _mlp_t(g4, W1t, b1c, W2t, b2c):
    return pl.pallas_call(
        _mlp_body,
        grid=(_L,),
        in_specs=[
            pl.BlockSpec((_B // 4, 128), lambda l: (l, 0)),
            pl.BlockSpec((_D_H, _D_E), lambda l: (0, 0)),
            pl.BlockSpec((_D_H, 1), lambda l: (0, 0)),
            pl.BlockSpec((_D_OUT, _D_H), lambda l: (0, 0)),
            pl.BlockSpec((_D_OUT, 1), lambda l: (0, 0)),
        ],
        out_specs=pl.BlockSpec((1, _D_OUT, _B), lambda l: (l, 0, 0)),
        out_shape=jax.ShapeDtypeStruct((_L, _D_OUT, _B), jnp.float32),
    )(g4, W1t, b1c, W2t, b2c)


def kernel(x, emb, W1, b1, W2, b2):
    idx = x.T.astype(jnp.int32)
    m = idx & (_PB - 1)
    idx2 = ((idx - m) + ((m & (_PQ - 1)) << 2) + (m >> 11)).reshape(
        _N // _CHUNK, _CHUNK)

    tab = _pack_table(emb.T).reshape(_VP, _D_E)
    g = _sc_gather(tab, idx2)
    g4 = g.reshape(_N // 4, 128)
    out_t = _mlp_t(g4, W1.T, b1.reshape(_D_H, 1), W2.T, b2.reshape(_D_OUT, 1))
    return out_t.transpose(2, 0, 1)


# speedup vs baseline: 45.0734x; 1.0139x over previous
import functools

import jax
import jax.numpy as jnp
from jax import lax
from jax.experimental import pallas as pl
from jax.experimental.pallas import tpu as pltpu
from jax.experimental.pallas import tpu_sc as plsc

_B = 16384
_L = 50
_D_E = 32
_D_H = 64
_D_OUT = 32
_N = _B * _L
_V = 1000000

_NC = 2
_NS = 16
_NW = _NC * _NS
_PER_W = _N // _NW
_CHUNK = 128
_NCHUNK = _PER_W // _CHUNK
_NBUF = 4
_NGROUP = _NCHUNK // _NBUF
_IROW_W = _PER_W // _CHUNK

_PB = 8192
_PQ = _PB // 4
_PG = (_V + _PB - 1) // _PB
_VP = _PG * _PB


def _pack_body(e_ref, eye_ref, o_ref):
    e = e_ref[...]
    eye = eye_ref[...]
    qs = [
        lax.dot_general(e[:, k * _PQ:(k + 1) * _PQ], eye,
                        (((0,), (0,)), ((), ())),
                        preferred_element_type=jnp.float32)
        for k in range(4)
    ]
    o_ref[...] = jnp.concatenate(qs, axis=1)


def _pack_table(emb_t):
    return pl.pallas_call(
        _pack_body,
        grid=(_PG,),
        in_specs=[
            pl.BlockSpec((_D_E, _PB), lambda c: (0, c)),
            pl.BlockSpec((_D_E, _D_E), lambda c: (0, 0)),
        ],
        out_specs=pl.BlockSpec((_PQ, 128), lambda c: (c, 0)),
        out_shape=jax.ShapeDtypeStruct((_VP // 4, 128), jnp.float32),
    )(emb_t, jnp.eye(_D_E, dtype=jnp.float32))


def _sc_gather_body(tab_hbm, idx_hbm, out_hbm, idx_v, rows_v, dsti, *sems):
    gs = sems[:_NBUF]
    ws = sems[_NBUF:]
    wid = lax.axis_index("s") * _NC + lax.axis_index("c")
    base = wid * _PER_W
    iv16 = lax.iota(jnp.int32, 16)

    pltpu.sync_copy(idx_hbm.at[pl.ds(wid * _IROW_W, _IROW_W)], idx_v)

    def g_copy(c, b):
        return pltpu.make_async_copy(
            tab_hbm.at[idx_v.at[c]], rows_v.at[b], gs[b])

    def w_copy(b):
        return pltpu.make_async_copy(
            rows_v.at[b], out_hbm.at[dsti.at[b]], ws[b])

    def fill_dst(c, b):
        t0 = base + c * _CHUNK
        d0 = ((t0 & ~16383) + ((t0 & 4095) << 2) + ((t0 >> 12) & 3))
        for v in range(8):
            dsti[b, pl.ds(v * 16, 16)] = d0 + 4 * (v * 16 + iv16)

    for b in range(_NBUF):
        g_copy(b, b).start()

    def body(g, carry):
        for b in range(_NBUF):
            c = g * _NBUF + b
            g_copy(c, b).wait()
            fill_dst(c, b)
            w_copy(b).start()

        @pl.when(g < _NGROUP - 1)
        def _prefetch():
            for b in range(_NBUF):
                c = (g + 1) * _NBUF + b
                w_copy(b).wait()
                g_copy(c, b).start()

        return carry

    lax.fori_loop(0, _NGROUP, body, 0)

    for b in range(_NBUF):
        w_copy(b).wait()


def _sc_gather(tab, idx2):
    mesh = plsc.VectorSubcoreMesh(
        core_axis_name="c", subcore_axis_name="s",
        num_cores=_NC, num_subcores=_NS)
    return pl.kernel(
        _sc_gather_body,
        out_type=jax.ShapeDtypeStruct((_N, _D_E), jnp.float32),
        mesh=mesh,
        scratch_types=[
            pltpu.VMEM((_NCHUNK, _CHUNK), jnp.int32),
            pltpu.VMEM((_NBUF, _CHUNK, _D_E), jnp.float32),
            pltpu.VMEM((_NBUF, _CHUNK), jnp.int32),
        ] + [pltpu.SemaphoreType.DMA] * (2 * _NBUF),
        compiler_params=pltpu.CompilerParams(use_tc_tiling_on_sc=False),
    )(tab, idx2)


_LB = 2


def _mlp_body(g_ref, w1t_ref, b1_ref, w2t_ref, b2_ref, o_ref):
    for s in range(_LB):
        e4t = g_ref[4096 * s:4096 * (s + 1), :].T
        for q in range(4):
            e = e4t[32 * q:32 * (q + 1), :]
            h = jnp.dot(w1t_ref[...], e, preferred_element_type=jnp.float32)
            h = jnp.maximum(h + b1_ref[...], 0.0)
            o = jnp.dot(w2t_ref[...], h, preferred_element_type=jnp.float32)
            o_ref[s, :, 4096 * q:4096 * (q + 1)] = o + b2_ref[...]


def _mlp_t(g4, W1t, b1c, W2t, b2c):
    return pl.pallas_call(
        _mlp_body,
        grid=(_L // _LB,),
        in_specs=[
            pl.BlockSpec((_LB * _B // 4, 128), lambda l: (l, 0)),
            pl.BlockSpec((_D_H, _D_E), lambda l: (0, 0)),
            pl.BlockSpec((_D_H, 1), lambda l: (0, 0)),
            pl.BlockSpec((_D_OUT, _D_H), lambda l: (0, 0)),
            pl.BlockSpec((_D_OUT, 1), lambda l: (0, 0)),
        ],
        out_specs=pl.BlockSpec((_LB, _D_OUT, _B), lambda l: (l, 0, 0)),
        out_shape=jax.ShapeDtypeStruct((_L, _D_OUT, _B), jnp.float32),
    )(g4, W1t, b1c, W2t, b2c)


def kernel(x, emb, W1, b1, W2, b2):
    idx = x.T.astype(jnp.int32)
    m = idx & (_PB - 1)
    idx2 = ((idx - m) + ((m & (_PQ - 1)) << 2) + (m >> 11)).reshape(
        _N // _CHUNK, _CHUNK)

    tab = _pack_table(emb.T).reshape(_VP, _D_E)
    g = _sc_gather(tab, idx2)
    g4 = g.reshape(_N // 4, 128)
    out_t = _mlp_t(g4, W1.T, b1.reshape(_D_H, 1), W2.T, b2.reshape(_D_OUT, 1))
    return out_t.transpose(2, 0, 1)


# speedup vs baseline: 51.7402x; 1.1479x over previous
import functools

import jax
import jax.numpy as jnp
from jax import lax
from jax.experimental import pallas as pl
from jax.experimental.pallas import tpu as pltpu
from jax.experimental.pallas import tpu_sc as plsc

_B = 16384
_L = 50
_D_E = 32
_D_H = 64
_D_OUT = 32
_N = _B * _L
_V = 1000000

_NC = 2
_NS = 16
_NW = _NC * _NS
_PER_W = _N // _NW
_CHUNK = 128
_NCHUNK = _PER_W // _CHUNK
_NBUF = 4
_NGROUP = _NCHUNK // _NBUF
_IROW_W = _PER_W // _CHUNK

_PB = 8192
_PQ = _PB // 4
_PG = (_V + _PB - 1) // _PB
_VP = _PG * _PB


def _pack_body(e_ref, eye_ref, o_ref):
    e = e_ref[...]
    w = eye_ref[...]
    acc = None
    for k in range(4):
        part = lax.dot_general(
            e[:, k * _PQ:(k + 1) * _PQ], w[32 * k:32 * (k + 1), :],
            (((0,), (0,)), ((), ())),
            preferred_element_type=jnp.float32)
        acc = part if acc is None else acc + part
    o_ref[...] = acc


def _pack_table(emb_t):
    return pl.pallas_call(
        _pack_body,
        grid=(_PG,),
        in_specs=[
            pl.BlockSpec((_D_E, _PB), lambda c: (0, c)),
            pl.BlockSpec((128, 128), lambda c: (0, 0)),
        ],
        out_specs=pl.BlockSpec((_PQ, 128), lambda c: (c, 0)),
        out_shape=jax.ShapeDtypeStruct((_VP // 4, 128), jnp.float32),
    )(emb_t, jnp.eye(128, dtype=jnp.float32))


def _sc_gather_body(tab_hbm, idx_hbm, out_hbm, idx_v, rows_v, dsti, *sems):
    gs = sems[:_NBUF]
    ws = sems[_NBUF:]
    wid = lax.axis_index("s") * _NC + lax.axis_index("c")
    base = wid * _PER_W
    iv16 = lax.iota(jnp.int32, 16)

    pltpu.sync_copy(idx_hbm.at[pl.ds(wid * _IROW_W, _IROW_W)], idx_v)

    def g_copy(c, b):
        return pltpu.make_async_copy(
            tab_hbm.at[idx_v.at[c]], rows_v.at[b], gs[b])

    def w_copy(b):
        return pltpu.make_async_copy(
            rows_v.at[b], out_hbm.at[dsti.at[b]], ws[b])

    def fill_dst(c, b):
        t0 = base + c * _CHUNK
        d0 = ((t0 & ~16383) + ((t0 & 4095) << 2) + ((t0 >> 12) & 3))
        for v in range(8):
            dsti[b, pl.ds(v * 16, 16)] = d0 + 4 * (v * 16 + iv16)

    for b in range(_NBUF):
        g_copy(b, b).start()

    def body(g, carry):
        for b in range(_NBUF):
            c = g * _NBUF + b
            g_copy(c, b).wait()
            fill_dst(c, b)
            w_copy(b).start()

        @pl.when(g < _NGROUP - 1)
        def _prefetch():
            for b in range(_NBUF):
                c = (g + 1) * _NBUF + b
                w_copy(b).wait()
                g_copy(c, b).start()

        return carry

    lax.fori_loop(0, _NGROUP, body, 0)

    for b in range(_NBUF):
        w_copy(b).wait()


def _sc_gather(tab, idx2):
    mesh = plsc.VectorSubcoreMesh(
        core_axis_name="c", subcore_axis_name="s",
        num_cores=_NC, num_subcores=_NS)
    return pl.kernel(
        _sc_gather_body,
        out_type=jax.ShapeDtypeStruct((_N, _D_E), jnp.float32),
        mesh=mesh,
        scratch_types=[
            pltpu.VMEM((_NCHUNK, _CHUNK), jnp.int32),
            pltpu.VMEM((_NBUF, _CHUNK, _D_E), jnp.float32),
            pltpu.VMEM((_NBUF, _CHUNK), jnp.int32),
        ] + [pltpu.SemaphoreType.DMA] * (2 * _NBUF),
        compiler_params=pltpu.CompilerParams(use_tc_tiling_on_sc=False),
    )(tab, idx2)


_LB = 2


def _mlp_body(g_ref, w1t_ref, b1_ref, w2t_ref, b2_ref, o_ref):
    for s in range(_LB):
        e4t = g_ref[4096 * s:4096 * (s + 1), :].T
        for q in range(4):
            e = e4t[32 * q:32 * (q + 1), :]
            h = jnp.dot(w1t_ref[...], e, preferred_element_type=jnp.float32)
            h = jnp.maximum(h + b1_ref[...], 0.0)
            o = jnp.dot(w2t_ref[...], h, preferred_element_type=jnp.float32)
            o_ref[s, :, 4096 * q:4096 * (q + 1)] = o + b2_ref[...]


def _mlp_t(g4, W1t, b1c, W2t, b2c):
    return pl.pallas_call(
        _mlp_body,
        grid=(_L // _LB,),
        in_specs=[
            pl.BlockSpec((_LB * _B // 4, 128), lambda l: (l, 0)),
            pl.BlockSpec((_D_H, _D_E), lambda l: (0, 0)),
            pl.BlockSpec((_D_H, 1), lambda l: (0, 0)),
            pl.BlockSpec((_D_OUT, _D_H), lambda l: (0, 0)),
            pl.BlockSpec((_D_OUT, 1), lambda l: (0, 0)),
        ],
        out_specs=pl.BlockSpec((_LB, _D_OUT, _B), lambda l: (l, 0, 0)),
        out_shape=jax.ShapeDtypeStruct((_L, _D_OUT, _B), jnp.float32),
    )(g4, W1t, b1c, W2t, b2c)


def kernel(x, emb, W1, b1, W2, b2):
    idx = x.T.astype(jnp.int32)
    m = idx & (_PB - 1)
    idx2 = ((idx - m) + ((m & (_PQ - 1)) << 2) + (m >> 11)).reshape(
        _N // _CHUNK, _CHUNK)

    tab = _pack_table(emb.T).reshape(_VP, _D_E)
    g = _sc_gather(tab, idx2)
    g4 = g.reshape(_N // 4, 128)
    out_t = _mlp_t(g4, W1.T, b1.reshape(_D_H, 1), W2.T, b2.reshape(_D_OUT, 1))
    return out_t.transpose(2, 0, 1)
